# baseline (speedup 1.0000x reference)
import functools

import jax
import jax.numpy as jnp
import numpy as np
from jax import lax
from jax.experimental import pallas as pl
from jax.experimental.pallas import tpu as pltpu
from jax.experimental.pallas import tpu_sc as plsc

K = 32
RADIUS = 0.1
N1 = 16384
N2 = 16384
G = 10
NCELLS = G * G * G
PG = G + 2
PROW = PG * PG
NSTARTS = 1744
QPAD = 16
NC = 2
NS = 16
NW = NC * NS
QPT = N1 // NW
GRP = 128
CAP = 96
COMPACT_AT = CAP - 16
SHIFT = 14
POSMASK = (1 << SHIFT) - 1
INF = (1 << 31) - 1
R2 = np.float32(RADIUS * RADIUS)


def _merge48(lo, hi, s):
    rs = lax.rev(s, (0,))
    m1 = jnp.minimum(hi, rs)
    p = jnp.minimum(lo, m1)
    q = jnp.maximum(lo, m1)
    return lax.sort(p), lax.sort(q)


def _smallest32(buf, bp, iota16, vinf):
    lo = vinf
    hi = vinf
    for j in range(CAP // 16):
        v = buf[pl.ds(16 * j, 16)]
        m = iota16 < (bp - 16 * j)
        v = jnp.where(m, v, vinf)
        lo, hi = _merge48(lo, hi, lax.sort(v))
    return lo, hi


def _smallest32_cond(buf, bp, iota16, vinf):
    lo = vinf
    hi = vinf
    for j in range(CAP // 16):
        def stage(args, j=j):
            a, b = args
            v = buf[pl.ds(16 * j, 16)]
            m = iota16 < (bp - 16 * j)
            v = jnp.where(m, v, vinf)
            return _merge48(a, b, lax.sort(v))

        lo, hi = lax.cond(bp > 16 * j, stage, lambda a: a, (lo, hi))
    return lo, hi


def _ballq_body(qx_h, qy_h, qz_h, sx_h, sy_h, sz_h, spk_h, starts_h,
                map_h, num_h, crd_h,
                sxv, syv, szv, spkv, stv, qxv, qyv, qzv,
                buf, rowbuf, mstage, cstage, nstage):
    wid = lax.axis_index("s") * NC + lax.axis_index("c")
    qbase = wid * QPT

    pltpu.sync_copy(sx_h, sxv)
    pltpu.sync_copy(sy_h, syv)
    pltpu.sync_copy(sz_h, szv)
    pltpu.sync_copy(spk_h, spkv.at[pl.ds(0, N2)])
    pltpu.sync_copy(starts_h, stv)
    pltpu.sync_copy(qx_h.at[pl.ds(qbase, QPT)], qxv.at[pl.ds(0, QPT)])
    pltpu.sync_copy(qy_h.at[pl.ds(qbase, QPT)], qyv.at[pl.ds(0, QPT)])
    pltpu.sync_copy(qz_h.at[pl.ds(qbase, QPT)], qzv.at[pl.ds(0, QPT)])

    iota16 = lax.broadcasted_iota(jnp.int32, (16,), 0)
    vinf = jnp.full((16,), INF, jnp.int32)

    def compact(bp):
        lo, hi = _smallest32(buf, bp, iota16, vinf)
        buf[pl.ds(0, 16)] = lo
        buf[pl.ds(16, 16)] = hi
        return jnp.minimum(bp, 32)

    def test_chunk(b, e, px, py, pz):
        lanem = iota16 < (e - b)
        oi = spkv[pl.ds(b, 16)] & POSMASK
        xs = plsc.load_gather(sxv, [oi])
        ys = plsc.load_gather(syv, [oi])
        zs = plsc.load_gather(szv, [oi])
        dx = xs - px
        dy = ys - py
        dz = zs - pz
        d2 = dx * dx + dy * dy + dz * dz
        hit = (d2 <= R2) & lanem
        n = jnp.sum(jnp.where(hit, 1, 0).astype(jnp.int32))
        return oi, hit, n

    def chunk2_body(i, bp, s, e, px, py, pz):
        b = s + i * 32
        pk0, h0, n0 = test_chunk(b, e, px, py, pz)
        pk1, h1, n1 = test_chunk(b + 16, e, px, py, pz)
        plsc.store_compressed(buf.at[pl.ds(bp, 16)], pk0, mask=h0)
        bp = bp + n0
        plsc.store_compressed(buf.at[pl.ds(bp, 16)], pk1, mask=h1)
        return bp + n1

    def chunk_guarded_body(i, bp, s, e, px, py, pz):
        pk, h, n = test_chunk(s + i * 16, e, px, py, pz)
        bp = lax.cond(bp > COMPACT_AT, compact, lambda v: v, bp)
        plsc.store_compressed(buf.at[pl.ds(bp, 16)], pk, mask=h)
        return bp + n

    def query_body(ql, _, g):
        lq = g * GRP + ql
        px = qxv[pl.ds(lq, 16)][0]
        py = qyv[pl.ds(lq, 16)][0]
        pz = qzv[pl.ds(lq, 16)][0]
        def cell(pf):
            t = pf.astype(jnp.int32)
            t = jnp.where(t.astype(jnp.float32) > pf, t - 1, t)
            return jnp.minimum(t, G - 1)

        cx = cell(px * jnp.float32(G))
        cy = cell(py * jnp.float32(G))
        cz = cell(pz * jnp.float32(G))
        b0 = (cz * PG + cy) * PG + cx
        svec = jnp.zeros((16,), jnp.int32)
        evec = svec
        for dz in range(3):
            w0 = stv[pl.ds(b0 + dz * PROW, 16)]
            w1 = stv[pl.ds(b0 + dz * PROW + 16, 16)]
            se = ((w0[0], w0[3]), (w0[12], w0[15]), (w1[8], w1[11]))
            for dy in range(3):
                r = dz * 3 + dy
                svec = jnp.where(iota16 == r, se[dy][0], svec)
                evec = jnp.where(iota16 == r, se[dy][1], evec)
        rowbuf[pl.ds(0, 16)] = svec
        rowbuf[pl.ds(16, 16)] = evec

        def row_body(zy, bp):
            s = rowbuf[pl.ds(zy, 16)][0]
            e = rowbuf[pl.ds(16 + zy, 16)][0]
            need = e - s

            def fast(bp):
                nch2 = (need + 31) >> 5
                return lax.fori_loop(
                    0, nch2,
                    functools.partial(chunk2_body, s=s, e=e, px=px, py=py, pz=pz),
                    bp)

            def guarded(bp):
                nch = (need + 15) >> 4
                return lax.fori_loop(
                    0, nch,
                    functools.partial(chunk_guarded_body, s=s, e=e,
                                      px=px, py=py, pz=pz),
                    bp)

            return lax.cond(bp + need > CAP, guarded, fast, bp)

        bp = lax.fori_loop(0, 9, row_body, jnp.int32(0))

        lo, hi = _smallest32_cond(buf, bp, iota16, vinf)
        num = jnp.minimum(bp, K)
        vlo = lo < INF
        vhi = hi < INF
        map_lo = jnp.where(vlo, lo, 0)
        map_hi = jnp.where(vhi, hi, 0)
        mstage[ql, pl.ds(0, 16)] = map_lo
        mstage[ql, pl.ds(16, 16)] = map_hi
        zf = jnp.float32(0.0)
        i3 = iota16 * 3
        crow = cstage.at[ql]
        gx = jnp.where(vlo, plsc.load_gather(sxv, [map_lo]), zf)
        gy = jnp.where(vlo, plsc.load_gather(syv, [map_lo]), zf)
        gz = jnp.where(vlo, plsc.load_gather(szv, [map_lo]), zf)
        plsc.store_scatter(crow, [i3], gx)
        plsc.store_scatter(crow, [i3 + 1], gy)
        plsc.store_scatter(crow, [i3 + 2], gz)
        gx = jnp.where(vhi, plsc.load_gather(sxv, [map_hi]), zf)
        gy = jnp.where(vhi, plsc.load_gather(syv, [map_hi]), zf)
        gz = jnp.where(vhi, plsc.load_gather(szv, [map_hi]), zf)
        plsc.store_scatter(crow, [i3 + 48], gx)
        plsc.store_scatter(crow, [i3 + 49], gy)
        plsc.store_scatter(crow, [i3 + 50], gz)
        plsc.store_scatter(nstage, [jnp.full((16,), ql, jnp.int32)],
                           jnp.full((16,), 1, jnp.int32) * num,
                           mask=iota16 == 0)
        return 0

    def group_body(g, _):
        lax.fori_loop(0, GRP, functools.partial(query_body, g=g), 0)
        off = qbase + g * GRP
        pltpu.sync_copy(mstage, map_h.at[pl.ds(off, GRP)])
        pltpu.sync_copy(cstage, crd_h.at[pl.ds(off, GRP)])
        pltpu.sync_copy(nstage, num_h.at[pl.ds(off, GRP)])
        return 0

    lax.fori_loop(0, QPT // GRP, group_body, 0)


@jax.jit
def _ballq(qx, qy, qz, sx, sy, sz, spk, starts):
    mesh = plsc.VectorSubcoreMesh(
        core_axis_name="c", subcore_axis_name="s",
        num_cores=NC, num_subcores=NS)
    f = pl.kernel(
        _ballq_body,
        out_type=(
            jax.ShapeDtypeStruct((N1, K), jnp.int32),
            jax.ShapeDtypeStruct((N1,), jnp.int32),
            jax.ShapeDtypeStruct((N1, 3 * K), jnp.float32),
        ),
        mesh=mesh,
        compiler_params=pltpu.CompilerParams(needs_layout_passes=False),
        scratch_types=(
            pltpu.VMEM((N2,), jnp.float32),
            pltpu.VMEM((N2,), jnp.float32),
            pltpu.VMEM((N2,), jnp.float32),
            pltpu.VMEM((N2 + 48,), jnp.int32),
            pltpu.VMEM((NSTARTS,), jnp.int32),
            pltpu.VMEM((QPT + QPAD,), jnp.float32),
            pltpu.VMEM((QPT + QPAD,), jnp.float32),
            pltpu.VMEM((QPT + QPAD,), jnp.float32),
            pltpu.VMEM((CAP + 16,), jnp.int32),
            pltpu.VMEM((48,), jnp.int32),
            pltpu.VMEM((GRP, K), jnp.int32),
            pltpu.VMEM((GRP, 3 * K), jnp.float32),
            pltpu.VMEM((GRP,), jnp.int32),
        ),
    )
    return f(qx, qy, qz, sx, sy, sz, spk, starts)


def kernel(points1, points2, lengths1, lengths2):
    p1 = points1[0]
    p2 = points2[0]
    qx, qy, qz = p1[:, 0], p1[:, 1], p1[:, 2]
    gx = jnp.minimum((p2[:, 0] * jnp.float32(G)).astype(jnp.int32), G - 1)
    gy = jnp.minimum((p2[:, 1] * jnp.float32(G)).astype(jnp.int32), G - 1)
    gz = jnp.minimum((p2[:, 2] * jnp.float32(G)).astype(jnp.int32), G - 1)
    pcid = ((gz + 1) * PG + (gy + 1)) * PG + (gx + 1)
    iota = jnp.arange(N2, dtype=jnp.int32)
    srt = jnp.sort((pcid << SHIFT) | iota)
    order = srt & POSMASK
    starts = jnp.searchsorted(
        srt, jnp.arange(NSTARTS, dtype=jnp.int32) << SHIFT, side="left"
    ).astype(jnp.int32)
    mapping, num, crd = _ballq(qx, qy, qz, p2[:, 0], p2[:, 1], p2[:, 2],
                               order, starts)
    return mapping[None], num[None], crd.reshape(N1, K, 3)[None]


# speedup vs baseline: 1.1013x; 1.1013x over previous
import functools

import jax
import jax.numpy as jnp
import numpy as np
from jax import lax
from jax.experimental import pallas as pl
from jax.experimental.pallas import tpu as pltpu
from jax.experimental.pallas import tpu_sc as plsc

K = 32
RADIUS = 0.1
N1 = 16384
N2 = 16384
G = 10
NCELLS = G * G * G
PG = G + 2
PROW = PG * PG
NSTARTS = 1744
QPAD = 16
NC = 2
NS = 16
NW = NC * NS
QPT = N1 // NW
GRP = 128
CAP = 96
COMPACT_AT = CAP - 16
SHIFT = 14
POSMASK = (1 << SHIFT) - 1
INF = (1 << 31) - 1
R2 = np.float32(RADIUS * RADIUS)


def _merge48(lo, hi, s):
    rs = lax.rev(s, (0,))
    m1 = jnp.minimum(hi, rs)
    p = jnp.minimum(lo, m1)
    q = jnp.maximum(lo, m1)
    return lax.sort(p), lax.sort(q)


def _smallest32(buf, bp, iota16, vinf):
    lo = vinf
    hi = vinf
    for j in range(CAP // 16):
        v = buf[pl.ds(16 * j, 16)]
        m = iota16 < (bp - 16 * j)
        v = jnp.where(m, v, vinf)
        lo, hi = _merge48(lo, hi, lax.sort(v))
    return lo, hi


def _smallest32_cond(buf, bp, iota16, vinf):
    lo = vinf
    hi = vinf
    for j in range(CAP // 16):
        def stage(args, j=j):
            a, b = args
            v = buf[pl.ds(16 * j, 16)]
            m = iota16 < (bp - 16 * j)
            v = jnp.where(m, v, vinf)
            return _merge48(a, b, lax.sort(v))

        lo, hi = lax.cond(bp > 16 * j, stage, lambda a: a, (lo, hi))
    return lo, hi


def _ballq_body(qx_h, qy_h, qz_h, sx_h, sy_h, sz_h, spk_h, starts_h,
                map_h, num_h, crd_h,
                sxv, syv, szv, spkv, stv, qxv, qyv, qzv,
                buf, rowbuf, mstage, cstage, nstage):
    wid = lax.axis_index("s") * NC + lax.axis_index("c")
    qbase = wid * QPT

    pltpu.sync_copy(sx_h, sxv.at[pl.ds(0, N2)])
    pltpu.sync_copy(sy_h, syv.at[pl.ds(0, N2)])
    pltpu.sync_copy(sz_h, szv.at[pl.ds(0, N2)])
    pltpu.sync_copy(spk_h, spkv.at[pl.ds(0, N2)])
    pltpu.sync_copy(starts_h, stv)
    pltpu.sync_copy(qx_h.at[pl.ds(qbase, QPT)], qxv.at[pl.ds(0, QPT)])
    pltpu.sync_copy(qy_h.at[pl.ds(qbase, QPT)], qyv.at[pl.ds(0, QPT)])
    pltpu.sync_copy(qz_h.at[pl.ds(qbase, QPT)], qzv.at[pl.ds(0, QPT)])

    iota16 = lax.broadcasted_iota(jnp.int32, (16,), 0)
    vinf = jnp.full((16,), INF, jnp.int32)

    def compact(bp):
        lo, hi = _smallest32(buf, bp, iota16, vinf)
        buf[pl.ds(0, 16)] = lo
        buf[pl.ds(16, 16)] = hi
        return jnp.minimum(bp, 32)

    def test_chunk(b, e, px, py, pz):
        lanem = iota16 < (e - b)
        xs = sxv[pl.ds(b, 16)]
        ys = syv[pl.ds(b, 16)]
        zs = szv[pl.ds(b, 16)]
        packed = spkv[pl.ds(b, 16)]
        dx = xs - px
        dy = ys - py
        dz = zs - pz
        d2 = dx * dx + dy * dy + dz * dz
        hit = (d2 <= R2) & lanem
        n = jnp.sum(jnp.where(hit, 1, 0).astype(jnp.int32))
        return packed, hit, n

    def chunk2_body(i, bp, s, e, px, py, pz):
        b = s + i * 32
        pk0, h0, n0 = test_chunk(b, e, px, py, pz)
        pk1, h1, n1 = test_chunk(b + 16, e, px, py, pz)
        plsc.store_compressed(buf.at[pl.ds(bp, 16)], pk0, mask=h0)
        bp = bp + n0
        plsc.store_compressed(buf.at[pl.ds(bp, 16)], pk1, mask=h1)
        return bp + n1

    def chunk_guarded_body(i, bp, s, e, px, py, pz):
        pk, h, n = test_chunk(s + i * 16, e, px, py, pz)
        bp = lax.cond(bp > COMPACT_AT, compact, lambda v: v, bp)
        plsc.store_compressed(buf.at[pl.ds(bp, 16)], pk, mask=h)
        return bp + n

    def query_body(ql, _, g):
        lq = g * GRP + ql
        px = qxv[pl.ds(lq, 16)][0]
        py = qyv[pl.ds(lq, 16)][0]
        pz = qzv[pl.ds(lq, 16)][0]
        def cell(pf):
            t = pf.astype(jnp.int32)
            t = jnp.where(t.astype(jnp.float32) > pf, t - 1, t)
            return jnp.minimum(t, G - 1)

        cx = cell(px * jnp.float32(G))
        cy = cell(py * jnp.float32(G))
        cz = cell(pz * jnp.float32(G))
        b0 = (cz * PG + cy) * PG + cx
        svec = jnp.zeros((16,), jnp.int32)
        evec = svec
        for dz in range(3):
            w0 = stv[pl.ds(b0 + dz * PROW, 16)]
            w1 = stv[pl.ds(b0 + dz * PROW + 16, 16)]
            se = ((w0[0], w0[3]), (w0[12], w0[15]), (w1[8], w1[11]))
            for dy in range(3):
                r = dz * 3 + dy
                svec = jnp.where(iota16 == r, se[dy][0], svec)
                evec = jnp.where(iota16 == r, se[dy][1], evec)
        rowbuf[pl.ds(0, 16)] = svec
        rowbuf[pl.ds(16, 16)] = evec

        def row_body(zy, bp):
            s = rowbuf[pl.ds(zy, 16)][0]
            e = rowbuf[pl.ds(16 + zy, 16)][0]
            need = e - s

            def fast(bp):
                nch2 = (need + 31) >> 5
                return lax.fori_loop(
                    0, nch2,
                    functools.partial(chunk2_body, s=s, e=e, px=px, py=py, pz=pz),
                    bp)

            def guarded(bp):
                nch = (need + 15) >> 4
                return lax.fori_loop(
                    0, nch,
                    functools.partial(chunk_guarded_body, s=s, e=e,
                                      px=px, py=py, pz=pz),
                    bp)

            return lax.cond(bp + need > CAP, guarded, fast, bp)

        bp = lax.fori_loop(0, 9, row_body, jnp.int32(0))

        lo, hi = _smallest32_cond(buf, bp, iota16, vinf)
        num = jnp.minimum(bp, K)
        vlo = lo < INF
        vhi = hi < INF
        map_lo = jnp.where(vlo, lo >> SHIFT, 0)
        map_hi = jnp.where(vhi, hi >> SHIFT, 0)
        pos_lo = jnp.where(vlo, lo & POSMASK, 0)
        pos_hi = jnp.where(vhi, hi & POSMASK, 0)
        mstage[ql, pl.ds(0, 16)] = map_lo
        mstage[ql, pl.ds(16, 16)] = map_hi
        zf = jnp.float32(0.0)
        i3 = iota16 * 3
        crow = cstage.at[ql]
        gx = jnp.where(vlo, plsc.load_gather(sxv, [pos_lo]), zf)
        gy = jnp.where(vlo, plsc.load_gather(syv, [pos_lo]), zf)
        gz = jnp.where(vlo, plsc.load_gather(szv, [pos_lo]), zf)
        plsc.store_scatter(crow, [i3], gx)
        plsc.store_scatter(crow, [i3 + 1], gy)
        plsc.store_scatter(crow, [i3 + 2], gz)
        gx = jnp.where(vhi, plsc.load_gather(sxv, [pos_hi]), zf)
        gy = jnp.where(vhi, plsc.load_gather(syv, [pos_hi]), zf)
        gz = jnp.where(vhi, plsc.load_gather(szv, [pos_hi]), zf)
        plsc.store_scatter(crow, [i3 + 48], gx)
        plsc.store_scatter(crow, [i3 + 49], gy)
        plsc.store_scatter(crow, [i3 + 50], gz)
        plsc.store_scatter(nstage, [jnp.full((16,), ql, jnp.int32)],
                           jnp.full((16,), 1, jnp.int32) * num,
                           mask=iota16 == 0)
        return 0

    def group_body(g, _):
        lax.fori_loop(0, GRP, functools.partial(query_body, g=g), 0)
        off = qbase + g * GRP
        pltpu.sync_copy(mstage, map_h.at[pl.ds(off, GRP)])
        pltpu.sync_copy(cstage, crd_h.at[pl.ds(off, GRP)])
        pltpu.sync_copy(nstage, num_h.at[pl.ds(off, GRP)])
        return 0

    lax.fori_loop(0, QPT // GRP, group_body, 0)


@jax.jit
def _ballq(qx, qy, qz, sx, sy, sz, spk, starts):
    mesh = plsc.VectorSubcoreMesh(
        core_axis_name="c", subcore_axis_name="s",
        num_cores=NC, num_subcores=NS)
    f = pl.kernel(
        _ballq_body,
        out_type=(
            jax.ShapeDtypeStruct((N1, K), jnp.int32),
            jax.ShapeDtypeStruct((N1,), jnp.int32),
            jax.ShapeDtypeStruct((N1, 3 * K), jnp.float32),
        ),
        mesh=mesh,
        compiler_params=pltpu.CompilerParams(needs_layout_passes=False),
        scratch_types=(
            pltpu.VMEM((N2 + 48,), jnp.float32),
            pltpu.VMEM((N2 + 48,), jnp.float32),
            pltpu.VMEM((N2 + 48,), jnp.float32),
            pltpu.VMEM((N2 + 48,), jnp.int32),
            pltpu.VMEM((NSTARTS,), jnp.int32),
            pltpu.VMEM((QPT + QPAD,), jnp.float32),
            pltpu.VMEM((QPT + QPAD,), jnp.float32),
            pltpu.VMEM((QPT + QPAD,), jnp.float32),
            pltpu.VMEM((CAP + 16,), jnp.int32),
            pltpu.VMEM((48,), jnp.int32),
            pltpu.VMEM((GRP, K), jnp.int32),
            pltpu.VMEM((GRP, 3 * K), jnp.float32),
            pltpu.VMEM((GRP,), jnp.int32),
        ),
    )
    return f(qx, qy, qz, sx, sy, sz, spk, starts)


def kernel(points1, points2, lengths1, lengths2):
    p1 = points1[0]
    p2 = points2[0]
    qx, qy, qz = p1[:, 0], p1[:, 1], p1[:, 2]
    gx = jnp.minimum((p2[:, 0] * jnp.float32(G)).astype(jnp.int32), G - 1)
    gy = jnp.minimum((p2[:, 1] * jnp.float32(G)).astype(jnp.int32), G - 1)
    gz = jnp.minimum((p2[:, 2] * jnp.float32(G)).astype(jnp.int32), G - 1)
    pcid = ((gz + 1) * PG + (gy + 1)) * PG + (gx + 1)
    iota = jnp.arange(N2, dtype=jnp.int32)
    srt = jnp.sort((pcid << SHIFT) | iota)
    order = srt & POSMASK
    spk = (order << SHIFT) | iota
    starts = jnp.searchsorted(
        srt, jnp.arange(NSTARTS, dtype=jnp.int32) << SHIFT, side="left"
    ).astype(jnp.int32)
    mapping, num, crd = _ballq(qx, qy, qz, p2[order, 0], p2[order, 1],
                               p2[order, 2], spk, starts)
    return mapping[None], num[None], crd.reshape(N1, K, 3)[None]
